# baseline (speedup 1.0000x reference)
import functools

import jax
import jax.numpy as jnp
from jax import lax
from jax.experimental import pallas as pl
from jax.experimental.pallas import tpu as pltpu
from jax.experimental.pallas import tpu_sc as plsc

F32 = jnp.float32
BF = jnp.bfloat16
_NC, _NS = 2, 16
_NW = _NC * _NS
_CH = 128
_NBLK = 1000
_EBLK = 2000


def _mmb(a, b):
    return jnp.dot(a.astype(BF), b.astype(BF), preferred_element_type=F32)


def _pad_n(N):
    q = _NS * 8
    return (N + q - 1) // q * q


def _node_enc_body(nL, nf, W, b, Wzs, Wzd, x_o, zs_o, zd_o):
    x = nf[...]
    for i in range(nL):
        x = _mmb(x, W[i]) + b[i:i + 1, :]
    x_o[...] = x
    zs_o[...] = _mmb(x, Wzs[...])
    zd_o[...] = _mmb(x, Wzd[...])


def _tc_node_enc(N, H, nf, W, b, Wzs, Wzd):
    grid = (N // _NBLK,)
    row = pl.BlockSpec((_NBLK, H), lambda i: (i, 0))
    full = lambda a: pl.BlockSpec(a.shape, lambda i: (0,) * a.ndim)
    return pl.pallas_call(
        functools.partial(_node_enc_body, W.shape[0]),
        grid=grid,
        in_specs=[pl.BlockSpec((_NBLK, nf.shape[1]), lambda i: (i, 0)),
                  full(W), full(b), full(Wzs), full(Wzd)],
        out_specs=[row, row, row],
        out_shape=[jax.ShapeDtypeStruct((N, H), F32)] * 3,
    )(nf, W, b, Wzs, Wzd)


def _edge_enc_body(nL, ef, W0, b0, W, b, e_o):
    e = _mmb(ef[...], W0[...]) + b0[...]
    for i in range(nL):
        e = _mmb(e, W[i]) + b[i:i + 1, :]
    e_o[...] = e


def _tc_edge_enc(E, H, ef, W0, b0, W, b):
    grid = (E // _EBLK,)
    full = lambda a: pl.BlockSpec(a.shape, lambda i: (0,) * a.ndim)
    return pl.pallas_call(
        functools.partial(_edge_enc_body, W.shape[0]),
        grid=grid,
        in_specs=[pl.BlockSpec((_EBLK, ef.shape[1]), lambda i: (i, 0)),
                  full(W0), full(b0), full(W), full(b)],
        out_specs=pl.BlockSpec((_EBLK, H), lambda i: (i, 0)),
        out_shape=jax.ShapeDtypeStruct((E, H), F32),
    )(ef, W0, b0, W, b)


def _edge_enc0_body(nLe, nL, ef, W0, b0, W, b, gsum, W0e, b0s, Ws, bs, m_o):
    e = _mmb(ef[...], W0[...]) + b0[...]
    for i in range(nLe):
        e = _mmb(e, W[i]) + b[i:i + 1, :]
    m = _mmb(e, W0e[...]) + gsum[...] + b0s[...]
    for i in range(nL):
        m = _mmb(m, Ws[i]) + bs[i:i + 1, :]
    m_o[...] = m


def _tc_edge_enc0(E, H, ef, W0, b0, W, b, gsum, W0e, b0s, Ws, bs):
    grid = (E // _EBLK,)
    row = pl.BlockSpec((_EBLK, H), lambda i: (i, 0))
    full = lambda a: pl.BlockSpec(a.shape, lambda i: (0,) * a.ndim)
    return pl.pallas_call(
        functools.partial(_edge_enc0_body, W.shape[0], Ws.shape[0]),
        grid=grid,
        in_specs=[pl.BlockSpec((_EBLK, ef.shape[1]), lambda i: (i, 0)),
                  full(W0), full(b0), full(W), full(b), row,
                  full(W0e), full(b0s), full(Ws), full(bs)],
        out_specs=row,
        out_shape=jax.ShapeDtypeStruct((E, H), F32),
    )(ef, W0, b0, W, b, gsum, W0e, b0s, Ws, bs)


_CHN = 80
_NBUF = 4


def _gsum_body(H, ncht, zs_h, zd_h, src3_h, dst3_h, g_out, junk_out,
               idxs, idxd, rows_a, rows_b, sem_a, sem_b, sem_w):
    cid = lax.axis_index("c")
    sid = lax.axis_index("s")
    wid = sid * _NC + cid
    ebase = wid * (ncht * _CHN)
    pltpu.sync_copy(src3_h.at[wid], idxs)
    pltpu.sync_copy(dst3_h.at[wid], idxd)

    def vadd(jp):
        @pl.loop(0, _CHN)
        def _(r):
            for c8 in range(H // 16):
                rows_a[jp, r, pl.ds(c8 * 16, 16)] = (
                    rows_a[jp, r, pl.ds(c8 * 16, 16)]
                    + rows_b[jp, r, pl.ds(c8 * 16, 16)])

    def gather(c, j):
        pltpu.async_copy(zs_h.at[idxs.at[c]], rows_a.at[j], sem_a[j])
        pltpu.async_copy(zd_h.at[idxd.at[c]], rows_b.at[j], sem_b[j])

    def wait_g(c, j):
        pltpu.make_async_copy(zs_h.at[idxs.at[c]], rows_a.at[j], sem_a[j]).wait()
        pltpu.make_async_copy(zd_h.at[idxd.at[c]], rows_b.at[j], sem_b[j]).wait()

    def wb(c, j):
        pltpu.async_copy(rows_a.at[j], g_out.at[pl.ds(ebase + c * _CHN, _CHN)],
                         sem_w[j])

    def wait_w(j):
        pltpu.make_async_copy(zs_h.at[pl.ds(0, _CHN)], rows_a.at[j],
                              sem_w[j]).wait()

    for j in range(_NBUF):
        pltpu.async_copy(rows_a.at[j], junk_out, sem_w[j])

    nloop = (ncht - 2) // _NBUF

    @pl.loop(0, nloop)
    def _(g):
        for j in range(_NBUF):
            c = g * _NBUF + j
            jp = (j + _NBUF - 2) % _NBUF
            wait_w(j)
            gather(c, j)

            @pl.when(c >= 2)
            def _():
                wait_g(c - 2, jp)
                vadd(jp)
                wb(c - 2, jp)

    for k in range(nloop * _NBUF, ncht):
        j = k % _NBUF
        wait_w(j)
        gather(k, j)
        jp = (j + _NBUF - 2) % _NBUF
        wait_g(k - 2, jp)
        vadd(jp)
        wb(k - 2, jp)
    for k in range(ncht - 2, ncht):
        jlast = k % _NBUF
        wait_g(k, jlast)
        vadd(jlast)
        wb(k, jlast)
    for j in range(_NBUF):
        wait_w(j)


def _sc_gsum(zs, zd, src3, dst3):
    NW, ncht, CHN = src3.shape
    E = NW * ncht * CHN
    H = zs.shape[1]
    mesh = plsc.VectorSubcoreMesh(core_axis_name="c", subcore_axis_name="s")
    g, _ = pl.kernel(
        functools.partial(_gsum_body, H, ncht),
        out_type=(jax.ShapeDtypeStruct((E, H), F32),
                  jax.ShapeDtypeStruct((_CHN, H), F32)),
        mesh=mesh,
        scratch_types=[
            pltpu.VMEM((ncht, CHN), jnp.int32),
            pltpu.VMEM((ncht, CHN), jnp.int32),
            pltpu.VMEM((_NBUF, _CHN, H), F32),
            pltpu.VMEM((_NBUF, _CHN, H), F32),
            [pltpu.SemaphoreType.DMA] * _NBUF,
            [pltpu.SemaphoreType.DMA] * _NBUF,
            [pltpu.SemaphoreType.DMA] * _NBUF,
        ],
    )(zs, zd, src3, dst3)
    return g


def _edge_step_body(nL, e, gsum, W0e, b0, W, b, m_o):
    m = _mmb(e[...], W0e[...]) + gsum[...] + b0[...]
    for i in range(nL):
        m = _mmb(m, W[i]) + b[i:i + 1, :]
    m_o[...] = m


def _tc_edge_step(E, H, e, gsum, W0e, b0, W, b):
    grid = (E // _EBLK,)
    row = pl.BlockSpec((_EBLK, H), lambda i: (i, 0))
    full = lambda a: pl.BlockSpec(a.shape, lambda i: (0,) * a.ndim)
    return pl.pallas_call(
        functools.partial(_edge_step_body, W.shape[0]),
        grid=grid,
        in_specs=[row, row, full(W0e), full(b0), full(W), full(b)],
        out_specs=row,
        out_shape=jax.ShapeDtypeStruct((E, H), F32),
    )(e, gsum, W0e, b0, W, b)


_SBUF = 3


def _scat_body(Np, H, ncht, m_h, dst3_h, zer_h, f_out,
               idxd, rows, f_sh, sem_l):
    cid = lax.axis_index("c")
    sid = lax.axis_index("s")
    wid = sid * _NC + cid
    rpt = Np // _NS
    ebase = wid * (ncht * _CHN)
    pltpu.sync_copy(dst3_h.at[wid], idxd)
    pltpu.sync_copy(zer_h, f_sh.at[pl.ds(sid * rpt, rpt)])
    plsc.subcore_barrier()

    def load(c, j):
        pltpu.async_copy(m_h.at[pl.ds(ebase + c * _CHN, _CHN)], rows.at[j],
                         sem_l[j])

    def wait_l(j):
        pltpu.make_async_copy(m_h.at[pl.ds(0, _CHN)], rows.at[j],
                              sem_l[j]).wait()

    load(0, 0)
    nloop = ncht // _SBUF

    @pl.loop(0, nloop)
    def _(g):
        for j in range(_SBUF):
            c = g * _SBUF + j
            jn = (j + 1) % _SBUF
            wait_l(j)
            cn = jnp.minimum(c + 1, ncht - 1)
            load(cn, jn)
            pltpu.sync_copy(rows.at[j], f_sh.at[idxd.at[c]], add=True)

    for k in range(nloop * _SBUF, ncht):
        j = k % _SBUF
        wait_l(j)
        load(min(k + 1, ncht - 1), (j + 1) % _SBUF)
        pltpu.sync_copy(rows.at[j], f_sh.at[idxd.at[k]], add=True)
    wait_l(ncht % _SBUF)

    plsc.subcore_barrier()
    row0 = cid * Np + sid * rpt
    pltpu.sync_copy(f_sh.at[pl.ds(sid * rpt, rpt)], f_out.at[pl.ds(row0, rpt)])


def _sc_scatter(N, m, dst3):
    NW, ncht, CHN = dst3.shape
    E, H = m.shape
    Np = _pad_n(N)
    rpt = Np // _NS
    zer = jnp.zeros((rpt, H), F32)
    mesh = plsc.VectorSubcoreMesh(core_axis_name="c", subcore_axis_name="s")
    f2 = pl.kernel(
        functools.partial(_scat_body, Np, H, ncht),
        out_type=jax.ShapeDtypeStruct((2 * Np, H), F32),
        mesh=mesh,
        scratch_types=[
            pltpu.VMEM((ncht, CHN), jnp.int32),
            pltpu.VMEM((_SBUF, _CHN, H), F32),
            pltpu.VMEM_SHARED((Np, H), F32),
            [pltpu.SemaphoreType.DMA] * _SBUF,
        ],
    )(m, dst3, zer)
    return f2.reshape(2, Np, H)


def _node_step_body(nL, x, aggp, Wnx, Wna, bn0, W, b, Wzs, Wzd,
                    x_o, zs_o, zd_o):
    agg = aggp[0] + aggp[1]
    h = _mmb(x[...], Wnx[...]) + _mmb(agg, Wna[...]) + bn0[...]
    for i in range(nL):
        h = _mmb(h, W[i]) + b[i:i + 1, :]
    x_o[...] = h
    zs_o[...] = _mmb(h, Wzs[...])
    zd_o[...] = _mmb(h, Wzd[...])


def _tc_node_step(N, H, x, aggp, Wnx, Wna, bn0, W, b, Wzs, Wzd):
    grid = (N // _NBLK,)
    row = pl.BlockSpec((_NBLK, H), lambda i: (i, 0))
    part = pl.BlockSpec((2, _NBLK, H), lambda i: (0, i, 0))
    full = lambda a: pl.BlockSpec(a.shape, lambda i: (0,) * a.ndim)
    return pl.pallas_call(
        functools.partial(_node_step_body, W.shape[0]),
        grid=grid,
        in_specs=[row, part, full(Wnx), full(Wna), full(bn0), full(W),
                  full(b), full(Wzs), full(Wzd)],
        out_specs=[row, row, row],
        out_shape=[jax.ShapeDtypeStruct((N, H), F32)] * 3,
    )(x, aggp, Wnx, Wna, bn0, W, b, Wzs, Wzd)


def _node_fin_body(nL, dL, x, aggp, Wnx, Wna, bn0, W, b, Wd, bd, out_o):
    agg = aggp[0] + aggp[1]
    h = _mmb(x[...], Wnx[...]) + _mmb(agg, Wna[...]) + bn0[...]
    for i in range(nL):
        h = _mmb(h, W[i]) + b[i:i + 1, :]
    for i in range(dL):
        h = _mmb(h, Wd[i]) + bd[i:i + 1, :]
    out_o[...] = h


def _tc_node_fin(N, H, Dn, x, aggp, Wnx, Wna, bn0, W, b, Wd, bd):
    grid = (N // _NBLK,)
    row = pl.BlockSpec((_NBLK, H), lambda i: (i, 0))
    part = pl.BlockSpec((2, _NBLK, H), lambda i: (0, i, 0))
    full = lambda a: pl.BlockSpec(a.shape, lambda i: (0,) * a.ndim)
    return pl.pallas_call(
        functools.partial(_node_fin_body, W.shape[0], Wd.shape[0]),
        grid=grid,
        in_specs=[row, part, full(Wnx), full(Wna), full(bn0), full(W),
                  full(b), full(Wd), full(bd)],
        out_specs=pl.BlockSpec((_NBLK, Dn), lambda i: (i, 0)),
        out_shape=jax.ShapeDtypeStruct((N, Dn), F32),
    )(x, aggp, Wnx, Wna, bn0, W, b, Wd, bd)


def kernel(node_features_in, edge_features_in, edges_indexes,
           enc_n_W, enc_n_b, enc_e_W0, enc_e_b0, enc_e_W, enc_e_b,
           proc_e_W0, proc_e_b0, proc_e_W, proc_e_b,
           proc_n_W0, proc_n_b0, proc_n_W, proc_n_b, dec_W, dec_b):
    N, Dn = node_features_in.shape
    E, De = edge_features_in.shape
    H = enc_n_W.shape[-1]
    S = proc_e_W0.shape[0]
    src = edges_indexes[0]
    dst = edges_indexes[1]

    ncht = E // (_NW * _CHN)
    assert E == _NW * ncht * _CHN
    src3 = src.reshape(_NW, ncht, _CHN)
    dst3 = dst.reshape(_NW, ncht, _CHN)

    x, zs, zd = _tc_node_enc(N, H, node_features_in, enc_n_W, enc_n_b,
                             proc_e_W0[0, H:2 * H], proc_e_W0[0, 2 * H:3 * H])
    e = _tc_edge_enc(E, H, edge_features_in, enc_e_W0,
                     enc_e_b0.reshape(1, H), enc_e_W, enc_e_b)
    for s in range(S):
        gsum = _sc_gsum(zs, zd, src3, dst3)
        e = _tc_edge_step(E, H, e, gsum, proc_e_W0[s, 0:H],
                          proc_e_b0[s].reshape(1, H), proc_e_W[s],
                          proc_e_b[s])
        aggp = _sc_scatter(N, e, dst3)
        if s + 1 < S:
            x, zs, zd = _tc_node_step(
                N, H, x, aggp, proc_n_W0[s, 0:H], proc_n_W0[s, H:2 * H],
                proc_n_b0[s].reshape(1, H), proc_n_W[s], proc_n_b[s],
                proc_e_W0[s + 1, H:2 * H], proc_e_W0[s + 1, 2 * H:3 * H])
        else:
            out = _tc_node_fin(
                N, H, Dn, x, aggp, proc_n_W0[s, 0:H], proc_n_W0[s, H:2 * H],
                proc_n_b0[s].reshape(1, H), proc_n_W[s], proc_n_b[s],
                dec_W, dec_b)
    return out


# speedup vs baseline: 1.2191x; 1.2191x over previous
import functools

import jax
import jax.numpy as jnp
from jax import lax
from jax.experimental import pallas as pl
from jax.experimental.pallas import tpu as pltpu
from jax.experimental.pallas import tpu_sc as plsc

F32 = jnp.float32
BF = jnp.bfloat16
_NC, _NS = 2, 16
_NW = _NC * _NS
_CH = 128
_NBLK = 1000
_EBLK = 4000


def _mmb(a, b):
    return jnp.dot(a.astype(BF), b.astype(BF), preferred_element_type=F32)


def _pad_n(N):
    q = _NS * 8
    return (N + q - 1) // q * q


def _node_enc_body(nL, nf, W, b, Wzs, Wzd, x_o, zs_o, zd_o):
    x = nf[...]
    for i in range(nL):
        x = _mmb(x, W[i]) + b[i:i + 1, :]
    x_o[...] = x
    zs_o[...] = _mmb(x, Wzs[...])
    zd_o[...] = _mmb(x, Wzd[...])


def _tc_node_enc(N, H, nf, W, b, Wzs, Wzd):
    grid = (N // _NBLK,)
    row = pl.BlockSpec((_NBLK, H), lambda i: (i, 0))
    full = lambda a: pl.BlockSpec(a.shape, lambda i: (0,) * a.ndim)
    return pl.pallas_call(
        functools.partial(_node_enc_body, W.shape[0]),
        grid=grid,
        in_specs=[pl.BlockSpec((_NBLK, nf.shape[1]), lambda i: (i, 0)),
                  full(W), full(b), full(Wzs), full(Wzd)],
        out_specs=[row, row, row],
        out_shape=[jax.ShapeDtypeStruct((N, H), F32)] * 3,
    )(nf, W, b, Wzs, Wzd)


def _edge_enc_body(nL, ef, W0, b0, W, b, e_o):
    e = _mmb(ef[...], W0[...]) + b0[...]
    for i in range(nL):
        e = _mmb(e, W[i]) + b[i:i + 1, :]
    e_o[...] = e


def _tc_edge_enc(E, H, ef, W0, b0, W, b):
    grid = (E // _EBLK,)
    full = lambda a: pl.BlockSpec(a.shape, lambda i: (0,) * a.ndim)
    return pl.pallas_call(
        functools.partial(_edge_enc_body, W.shape[0]),
        grid=grid,
        in_specs=[pl.BlockSpec((_EBLK, ef.shape[1]), lambda i: (i, 0)),
                  full(W0), full(b0), full(W), full(b)],
        out_specs=pl.BlockSpec((_EBLK, H), lambda i: (i, 0)),
        out_shape=jax.ShapeDtypeStruct((E, H), F32),
    )(ef, W0, b0, W, b)


def _edge_enc0_body(nLe, nL, ef, W0, b0, W, b, gsum, W0e, b0s, Ws, bs, m_o):
    e = _mmb(ef[...], W0[...]) + b0[...]
    for i in range(nLe):
        e = _mmb(e, W[i]) + b[i:i + 1, :]
    m = _mmb(e, W0e[...]) + gsum[...] + b0s[...]
    for i in range(nL):
        m = _mmb(m, Ws[i]) + bs[i:i + 1, :]
    m_o[...] = m


def _tc_edge_enc0(E, H, ef, W0, b0, W, b, gsum, W0e, b0s, Ws, bs):
    grid = (E // _EBLK,)
    row = pl.BlockSpec((_EBLK, H), lambda i: (i, 0))
    full = lambda a: pl.BlockSpec(a.shape, lambda i: (0,) * a.ndim)
    return pl.pallas_call(
        functools.partial(_edge_enc0_body, W.shape[0], Ws.shape[0]),
        grid=grid,
        in_specs=[pl.BlockSpec((_EBLK, ef.shape[1]), lambda i: (i, 0)),
                  full(W0), full(b0), full(W), full(b), row,
                  full(W0e), full(b0s), full(Ws), full(bs)],
        out_specs=row,
        out_shape=jax.ShapeDtypeStruct((E, H), F32),
    )(ef, W0, b0, W, b, gsum, W0e, b0s, Ws, bs)


_CHN = 80
_NBUF = 4


def _gsum_body(H, ncht, zs_h, zd_h, src3_h, dst3_h, g_out, junk_out,
               idxs, idxd, rows_a, rows_b, sem_a, sem_b, sem_w):
    cid = lax.axis_index("c")
    sid = lax.axis_index("s")
    wid = sid * _NC + cid
    ebase = wid * (ncht * _CHN)
    pltpu.sync_copy(src3_h.at[wid], idxs)
    pltpu.sync_copy(dst3_h.at[wid], idxd)

    def vadd(jp):
        @pl.loop(0, _CHN)
        def _(r):
            for c8 in range(H // 16):
                rows_a[jp, r, pl.ds(c8 * 16, 16)] = (
                    rows_a[jp, r, pl.ds(c8 * 16, 16)]
                    + rows_b[jp, r, pl.ds(c8 * 16, 16)])

    def gather(c, j):
        pltpu.async_copy(zs_h.at[idxs.at[c]], rows_a.at[j], sem_a[j])
        pltpu.async_copy(zd_h.at[idxd.at[c]], rows_b.at[j], sem_b[j])

    def wait_g(c, j):
        pltpu.make_async_copy(zs_h.at[idxs.at[c]], rows_a.at[j], sem_a[j]).wait()
        pltpu.make_async_copy(zd_h.at[idxd.at[c]], rows_b.at[j], sem_b[j]).wait()

    def wb(c, j):
        pltpu.async_copy(rows_a.at[j], g_out.at[pl.ds(ebase + c * _CHN, _CHN)],
                         sem_w[j])

    def wait_w(j):
        pltpu.make_async_copy(zs_h.at[pl.ds(0, _CHN)], rows_a.at[j],
                              sem_w[j]).wait()

    for j in range(_NBUF):
        pltpu.async_copy(rows_a.at[j], junk_out, sem_w[j])

    nloop = (ncht - 2) // _NBUF

    @pl.loop(0, nloop)
    def _(g):
        for j in range(_NBUF):
            c = g * _NBUF + j
            jp = (j + _NBUF - 2) % _NBUF
            wait_w(j)
            gather(c, j)

            @pl.when(c >= 2)
            def _():
                wait_g(c - 2, jp)
                vadd(jp)
                wb(c - 2, jp)

    for k in range(nloop * _NBUF, ncht):
        j = k % _NBUF
        wait_w(j)
        gather(k, j)
        jp = (j + _NBUF - 2) % _NBUF
        wait_g(k - 2, jp)
        vadd(jp)
        wb(k - 2, jp)
    for k in range(ncht - 2, ncht):
        jlast = k % _NBUF
        wait_g(k, jlast)
        vadd(jlast)
        wb(k, jlast)
    for j in range(_NBUF):
        wait_w(j)


def _sc_gsum(zs, zd, src3, dst3):
    NW, ncht, CHN = src3.shape
    E = NW * ncht * CHN
    H = zs.shape[1]
    mesh = plsc.VectorSubcoreMesh(core_axis_name="c", subcore_axis_name="s")
    g, _ = pl.kernel(
        functools.partial(_gsum_body, H, ncht),
        out_type=(jax.ShapeDtypeStruct((E, H), F32),
                  jax.ShapeDtypeStruct((_CHN, H), F32)),
        mesh=mesh,
        scratch_types=[
            pltpu.VMEM((ncht, CHN), jnp.int32),
            pltpu.VMEM((ncht, CHN), jnp.int32),
            pltpu.VMEM((_NBUF, _CHN, H), F32),
            pltpu.VMEM((_NBUF, _CHN, H), F32),
            [pltpu.SemaphoreType.DMA] * _NBUF,
            [pltpu.SemaphoreType.DMA] * _NBUF,
            [pltpu.SemaphoreType.DMA] * _NBUF,
        ],
    )(zs, zd, src3, dst3)
    return g


def _edge_step_body(nL, e, gsum, W0e, b0, W, b, m_o):
    m = _mmb(e[...], W0e[...]) + gsum[...] + b0[...]
    for i in range(nL):
        m = _mmb(m, W[i]) + b[i:i + 1, :]
    m_o[...] = m


def _tc_edge_step(E, H, e, gsum, W0e, b0, W, b):
    grid = (E // _EBLK,)
    row = pl.BlockSpec((_EBLK, H), lambda i: (i, 0))
    full = lambda a: pl.BlockSpec(a.shape, lambda i: (0,) * a.ndim)
    return pl.pallas_call(
        functools.partial(_edge_step_body, W.shape[0]),
        grid=grid,
        in_specs=[row, row, full(W0e), full(b0), full(W), full(b)],
        out_specs=row,
        out_shape=jax.ShapeDtypeStruct((E, H), F32),
    )(e, gsum, W0e, b0, W, b)


_SBUF = 3


def _scat_body(Np, H, ncht, m_h, dst3_h, zer_h, f_out,
               idxd, rows, f_sh, sem_l):
    cid = lax.axis_index("c")
    sid = lax.axis_index("s")
    wid = sid * _NC + cid
    rpt = Np // _NS
    ebase = wid * (ncht * _CHN)
    pltpu.sync_copy(dst3_h.at[wid], idxd)
    pltpu.sync_copy(zer_h, f_sh.at[pl.ds(sid * rpt, rpt)])
    plsc.subcore_barrier()

    def load(c, j):
        pltpu.async_copy(m_h.at[pl.ds(ebase + c * _CHN, _CHN)], rows.at[j],
                         sem_l[j])

    def wait_l(j):
        pltpu.make_async_copy(m_h.at[pl.ds(0, _CHN)], rows.at[j],
                              sem_l[j]).wait()

    load(0, 0)
    nloop = ncht // _SBUF

    @pl.loop(0, nloop)
    def _(g):
        for j in range(_SBUF):
            c = g * _SBUF + j
            jn = (j + 1) % _SBUF
            wait_l(j)
            cn = jnp.minimum(c + 1, ncht - 1)
            load(cn, jn)
            pltpu.sync_copy(rows.at[j], f_sh.at[idxd.at[c]], add=True)

    for k in range(nloop * _SBUF, ncht):
        j = k % _SBUF
        wait_l(j)
        load(min(k + 1, ncht - 1), (j + 1) % _SBUF)
        pltpu.sync_copy(rows.at[j], f_sh.at[idxd.at[k]], add=True)
    wait_l(ncht % _SBUF)

    plsc.subcore_barrier()
    row0 = cid * Np + sid * rpt
    pltpu.sync_copy(f_sh.at[pl.ds(sid * rpt, rpt)], f_out.at[pl.ds(row0, rpt)])


def _sc_scatter(N, m, dst3):
    NW, ncht, CHN = dst3.shape
    E, H = m.shape
    Np = _pad_n(N)
    rpt = Np // _NS
    zer = jnp.zeros((rpt, H), F32)
    mesh = plsc.VectorSubcoreMesh(core_axis_name="c", subcore_axis_name="s")
    f2 = pl.kernel(
        functools.partial(_scat_body, Np, H, ncht),
        out_type=jax.ShapeDtypeStruct((2 * Np, H), F32),
        mesh=mesh,
        scratch_types=[
            pltpu.VMEM((ncht, CHN), jnp.int32),
            pltpu.VMEM((_SBUF, _CHN, H), F32),
            pltpu.VMEM_SHARED((Np, H), F32),
            [pltpu.SemaphoreType.DMA] * _SBUF,
        ],
    )(m, dst3, zer)
    return f2.reshape(2, Np, H)


def _node_step_body(nL, x, aggp, Wnx, Wna, bn0, W, b, Wzs, Wzd,
                    x_o, zs_o, zd_o):
    agg = aggp[0] + aggp[1]
    h = _mmb(x[...], Wnx[...]) + _mmb(agg, Wna[...]) + bn0[...]
    for i in range(nL):
        h = _mmb(h, W[i]) + b[i:i + 1, :]
    x_o[...] = h
    zs_o[...] = _mmb(h, Wzs[...])
    zd_o[...] = _mmb(h, Wzd[...])


def _tc_node_step(N, H, x, aggp, Wnx, Wna, bn0, W, b, Wzs, Wzd):
    grid = (N // _NBLK,)
    row = pl.BlockSpec((_NBLK, H), lambda i: (i, 0))
    part = pl.BlockSpec((2, _NBLK, H), lambda i: (0, i, 0))
    full = lambda a: pl.BlockSpec(a.shape, lambda i: (0,) * a.ndim)
    return pl.pallas_call(
        functools.partial(_node_step_body, W.shape[0]),
        grid=grid,
        in_specs=[row, part, full(Wnx), full(Wna), full(bn0), full(W),
                  full(b), full(Wzs), full(Wzd)],
        out_specs=[row, row, row],
        out_shape=[jax.ShapeDtypeStruct((N, H), F32)] * 3,
    )(x, aggp, Wnx, Wna, bn0, W, b, Wzs, Wzd)


def _node_fin_body(nL, dL, x, aggp, Wnx, Wna, bn0, W, b, Wd, bd, out_o):
    agg = aggp[0] + aggp[1]
    h = _mmb(x[...], Wnx[...]) + _mmb(agg, Wna[...]) + bn0[...]
    for i in range(nL):
        h = _mmb(h, W[i]) + b[i:i + 1, :]
    for i in range(dL):
        h = _mmb(h, Wd[i]) + bd[i:i + 1, :]
    out_o[...] = h


def _tc_node_fin(N, H, Dn, x, aggp, Wnx, Wna, bn0, W, b, Wd, bd):
    grid = (N // _NBLK,)
    row = pl.BlockSpec((_NBLK, H), lambda i: (i, 0))
    part = pl.BlockSpec((2, _NBLK, H), lambda i: (0, i, 0))
    full = lambda a: pl.BlockSpec(a.shape, lambda i: (0,) * a.ndim)
    return pl.pallas_call(
        functools.partial(_node_fin_body, W.shape[0], Wd.shape[0]),
        grid=grid,
        in_specs=[row, part, full(Wnx), full(Wna), full(bn0), full(W),
                  full(b), full(Wd), full(bd)],
        out_specs=pl.BlockSpec((_NBLK, Dn), lambda i: (i, 0)),
        out_shape=jax.ShapeDtypeStruct((N, Dn), F32),
    )(x, aggp, Wnx, Wna, bn0, W, b, Wd, bd)


def kernel(node_features_in, edge_features_in, edges_indexes,
           enc_n_W, enc_n_b, enc_e_W0, enc_e_b0, enc_e_W, enc_e_b,
           proc_e_W0, proc_e_b0, proc_e_W, proc_e_b,
           proc_n_W0, proc_n_b0, proc_n_W, proc_n_b, dec_W, dec_b):
    N, Dn = node_features_in.shape
    E, De = edge_features_in.shape
    H = enc_n_W.shape[-1]
    S = proc_e_W0.shape[0]
    src = edges_indexes[0]
    dst = edges_indexes[1]

    ncht = E // (_NW * _CHN)
    assert E == _NW * ncht * _CHN
    src3 = src.reshape(_NW, ncht, _CHN)
    dst3 = dst.reshape(_NW, ncht, _CHN)

    x, zs, zd = _tc_node_enc(N, H, node_features_in, enc_n_W, enc_n_b,
                             proc_e_W0[0, H:2 * H], proc_e_W0[0, 2 * H:3 * H])
    for s in range(S):
        gsum = _sc_gsum(zs, zd, src3, dst3)
        if s == 0:
            e = _tc_edge_enc0(E, H, edge_features_in, enc_e_W0,
                              enc_e_b0.reshape(1, H), enc_e_W, enc_e_b,
                              gsum, proc_e_W0[0, 0:H],
                              proc_e_b0[0].reshape(1, H), proc_e_W[0],
                              proc_e_b[0])
        else:
            e = _tc_edge_step(E, H, e, gsum, proc_e_W0[s, 0:H],
                              proc_e_b0[s].reshape(1, H), proc_e_W[s],
                              proc_e_b[s])
        aggp = _sc_scatter(N, e, dst3)
        if s + 1 < S:
            x, zs, zd = _tc_node_step(
                N, H, x, aggp, proc_n_W0[s, 0:H], proc_n_W0[s, H:2 * H],
                proc_n_b0[s].reshape(1, H), proc_n_W[s], proc_n_b[s],
                proc_e_W0[s + 1, H:2 * H], proc_e_W0[s + 1, 2 * H:3 * H])
        else:
            out = _tc_node_fin(
                N, H, Dn, x, aggp, proc_n_W0[s, 0:H], proc_n_W0[s, H:2 * H],
                proc_n_b0[s].reshape(1, H), proc_n_W[s], proc_n_b[s],
                dec_W, dec_b)
    return out


# speedup vs baseline: 1.2657x; 1.0382x over previous
import functools

import jax
import jax.numpy as jnp
from jax import lax
from jax.experimental import pallas as pl
from jax.experimental.pallas import tpu as pltpu
from jax.experimental.pallas import tpu_sc as plsc

F32 = jnp.float32
BF = jnp.bfloat16
_NC, _NS = 2, 16
_NW = _NC * _NS
_CH = 128
_NBLK = 2000
_EBLK = 8000


def _mmb(a, b):
    return jnp.dot(a.astype(BF), b.astype(BF), preferred_element_type=F32)


def _pad_n(N):
    q = _NS * 8
    return (N + q - 1) // q * q


def _node_enc_body(nL, nf, W, b, Wzs, Wzd, x_o, zs_o, zd_o):
    x = nf[...]
    for i in range(nL):
        x = _mmb(x, W[i]) + b[i:i + 1, :]
    x_o[...] = x
    zs_o[...] = _mmb(x, Wzs[...])
    zd_o[...] = _mmb(x, Wzd[...])


def _tc_node_enc(N, H, nf, W, b, Wzs, Wzd):
    grid = (N // _NBLK,)
    row = pl.BlockSpec((_NBLK, H), lambda i: (i, 0))
    full = lambda a: pl.BlockSpec(a.shape, lambda i: (0,) * a.ndim)
    return pl.pallas_call(
        functools.partial(_node_enc_body, W.shape[0]),
        grid=grid,
        in_specs=[pl.BlockSpec((_NBLK, nf.shape[1]), lambda i: (i, 0)),
                  full(W), full(b), full(Wzs), full(Wzd)],
        out_specs=[row, row, row],
        out_shape=[jax.ShapeDtypeStruct((N, H), F32)] * 3,
    )(nf, W, b, Wzs, Wzd)


def _edge_enc_body(nL, ef, W0, b0, W, b, e_o):
    e = _mmb(ef[...], W0[...]) + b0[...]
    for i in range(nL):
        e = _mmb(e, W[i]) + b[i:i + 1, :]
    e_o[...] = e


def _tc_edge_enc(E, H, ef, W0, b0, W, b):
    grid = (E // _EBLK,)
    full = lambda a: pl.BlockSpec(a.shape, lambda i: (0,) * a.ndim)
    return pl.pallas_call(
        functools.partial(_edge_enc_body, W.shape[0]),
        grid=grid,
        in_specs=[pl.BlockSpec((_EBLK, ef.shape[1]), lambda i: (i, 0)),
                  full(W0), full(b0), full(W), full(b)],
        out_specs=pl.BlockSpec((_EBLK, H), lambda i: (i, 0)),
        out_shape=jax.ShapeDtypeStruct((E, H), F32),
    )(ef, W0, b0, W, b)


def _edge_enc0_body(nLe, nL, ef, W0, b0, W, b, gsum, W0e, b0s, Ws, bs, m_o):
    e = _mmb(ef[...], W0[...]) + b0[...]
    for i in range(nLe):
        e = _mmb(e, W[i]) + b[i:i + 1, :]
    m = _mmb(e, W0e[...]) + gsum[...] + b0s[...]
    for i in range(nL):
        m = _mmb(m, Ws[i]) + bs[i:i + 1, :]
    m_o[...] = m


def _tc_edge_enc0(E, H, ef, W0, b0, W, b, gsum, W0e, b0s, Ws, bs):
    grid = (E // _EBLK,)
    row = pl.BlockSpec((_EBLK, H), lambda i: (i, 0))
    full = lambda a: pl.BlockSpec(a.shape, lambda i: (0,) * a.ndim)
    return pl.pallas_call(
        functools.partial(_edge_enc0_body, W.shape[0], Ws.shape[0]),
        grid=grid,
        in_specs=[pl.BlockSpec((_EBLK, ef.shape[1]), lambda i: (i, 0)),
                  full(W0), full(b0), full(W), full(b), row,
                  full(W0e), full(b0s), full(Ws), full(bs)],
        out_specs=row,
        out_shape=jax.ShapeDtypeStruct((E, H), F32),
    )(ef, W0, b0, W, b, gsum, W0e, b0s, Ws, bs)


_CHN = 80
_NBUF = 4


def _gsum_body(H, ncht, zs_h, zd_h, src3_h, dst3_h, g_out, junk_out,
               idxs, idxd, rows_a, rows_b, sem_a, sem_b, sem_w):
    cid = lax.axis_index("c")
    sid = lax.axis_index("s")
    wid = sid * _NC + cid
    ebase = wid * (ncht * _CHN)
    pltpu.sync_copy(src3_h.at[wid], idxs)
    pltpu.sync_copy(dst3_h.at[wid], idxd)

    def vadd(jp):
        @pl.loop(0, _CHN)
        def _(r):
            for c8 in range(H // 16):
                rows_a[jp, r, pl.ds(c8 * 16, 16)] = (
                    rows_a[jp, r, pl.ds(c8 * 16, 16)]
                    + rows_b[jp, r, pl.ds(c8 * 16, 16)])

    def gather(c, j):
        pltpu.async_copy(zs_h.at[idxs.at[c]], rows_a.at[j], sem_a[j])
        pltpu.async_copy(zd_h.at[idxd.at[c]], rows_b.at[j], sem_b[j])

    def wait_g(c, j):
        pltpu.make_async_copy(zs_h.at[idxs.at[c]], rows_a.at[j], sem_a[j]).wait()
        pltpu.make_async_copy(zd_h.at[idxd.at[c]], rows_b.at[j], sem_b[j]).wait()

    def wb(c, j):
        pltpu.async_copy(rows_a.at[j], g_out.at[pl.ds(ebase + c * _CHN, _CHN)],
                         sem_w[j])

    def wait_w(j):
        pltpu.make_async_copy(zs_h.at[pl.ds(0, _CHN)], rows_a.at[j],
                              sem_w[j]).wait()

    for j in range(_NBUF):
        pltpu.async_copy(rows_a.at[j], junk_out, sem_w[j])

    nloop = (ncht - 2) // _NBUF

    @pl.loop(0, nloop)
    def _(g):
        for j in range(_NBUF):
            c = g * _NBUF + j
            jp = (j + _NBUF - 2) % _NBUF
            wait_w(j)
            gather(c, j)

            @pl.when(c >= 2)
            def _():
                wait_g(c - 2, jp)
                vadd(jp)
                wb(c - 2, jp)

    for k in range(nloop * _NBUF, ncht):
        j = k % _NBUF
        wait_w(j)
        gather(k, j)
        jp = (j + _NBUF - 2) % _NBUF
        wait_g(k - 2, jp)
        vadd(jp)
        wb(k - 2, jp)
    for k in range(ncht - 2, ncht):
        jlast = k % _NBUF
        wait_g(k, jlast)
        vadd(jlast)
        wb(k, jlast)
    for j in range(_NBUF):
        wait_w(j)


def _sc_gsum(zs, zd, src3, dst3):
    NW, ncht, CHN = src3.shape
    E = NW * ncht * CHN
    H = zs.shape[1]
    mesh = plsc.VectorSubcoreMesh(core_axis_name="c", subcore_axis_name="s")
    g, _ = pl.kernel(
        functools.partial(_gsum_body, H, ncht),
        out_type=(jax.ShapeDtypeStruct((E, H), F32),
                  jax.ShapeDtypeStruct((_CHN, H), F32)),
        mesh=mesh,
        scratch_types=[
            pltpu.VMEM((ncht, CHN), jnp.int32),
            pltpu.VMEM((ncht, CHN), jnp.int32),
            pltpu.VMEM((_NBUF, _CHN, H), F32),
            pltpu.VMEM((_NBUF, _CHN, H), F32),
            [pltpu.SemaphoreType.DMA] * _NBUF,
            [pltpu.SemaphoreType.DMA] * _NBUF,
            [pltpu.SemaphoreType.DMA] * _NBUF,
        ],
    )(zs, zd, src3, dst3)
    return g


def _edge_step_body(nL, e, gsum, W0e, b0, W, b, m_o):
    m = _mmb(e[...], W0e[...]) + gsum[...] + b0[...]
    for i in range(nL):
        m = _mmb(m, W[i]) + b[i:i + 1, :]
    m_o[...] = m


def _tc_edge_step(E, H, e, gsum, W0e, b0, W, b):
    grid = (E // _EBLK,)
    row = pl.BlockSpec((_EBLK, H), lambda i: (i, 0))
    full = lambda a: pl.BlockSpec(a.shape, lambda i: (0,) * a.ndim)
    return pl.pallas_call(
        functools.partial(_edge_step_body, W.shape[0]),
        grid=grid,
        in_specs=[row, row, full(W0e), full(b0), full(W), full(b)],
        out_specs=row,
        out_shape=jax.ShapeDtypeStruct((E, H), F32),
    )(e, gsum, W0e, b0, W, b)


_SBUF = 3


def _scat_body(Np, H, ncht, m_h, dst3_h, zer_h, f_out,
               idxd, rows, f_sh, sem_l):
    cid = lax.axis_index("c")
    sid = lax.axis_index("s")
    wid = sid * _NC + cid
    rpt = Np // _NS
    ebase = wid * (ncht * _CHN)
    pltpu.sync_copy(dst3_h.at[wid], idxd)
    pltpu.sync_copy(zer_h, f_sh.at[pl.ds(sid * rpt, rpt)])
    plsc.subcore_barrier()

    def load(c, j):
        pltpu.async_copy(m_h.at[pl.ds(ebase + c * _CHN, _CHN)], rows.at[j],
                         sem_l[j])

    def wait_l(j):
        pltpu.make_async_copy(m_h.at[pl.ds(0, _CHN)], rows.at[j],
                              sem_l[j]).wait()

    load(0, 0)
    nloop = ncht // _SBUF

    @pl.loop(0, nloop)
    def _(g):
        for j in range(_SBUF):
            c = g * _SBUF + j
            jn = (j + 1) % _SBUF
            wait_l(j)
            cn = jnp.minimum(c + 1, ncht - 1)
            load(cn, jn)
            pltpu.sync_copy(rows.at[j], f_sh.at[idxd.at[c]], add=True)

    for k in range(nloop * _SBUF, ncht):
        j = k % _SBUF
        wait_l(j)
        load(min(k + 1, ncht - 1), (j + 1) % _SBUF)
        pltpu.sync_copy(rows.at[j], f_sh.at[idxd.at[k]], add=True)
    wait_l(ncht % _SBUF)

    plsc.subcore_barrier()
    row0 = cid * Np + sid * rpt
    pltpu.sync_copy(f_sh.at[pl.ds(sid * rpt, rpt)], f_out.at[pl.ds(row0, rpt)])


def _sc_scatter(N, m, dst3):
    NW, ncht, CHN = dst3.shape
    E, H = m.shape
    Np = _pad_n(N)
    rpt = Np // _NS
    zer = jnp.zeros((rpt, H), F32)
    mesh = plsc.VectorSubcoreMesh(core_axis_name="c", subcore_axis_name="s")
    f2 = pl.kernel(
        functools.partial(_scat_body, Np, H, ncht),
        out_type=jax.ShapeDtypeStruct((2 * Np, H), F32),
        mesh=mesh,
        scratch_types=[
            pltpu.VMEM((ncht, CHN), jnp.int32),
            pltpu.VMEM((_SBUF, _CHN, H), F32),
            pltpu.VMEM_SHARED((Np, H), F32),
            [pltpu.SemaphoreType.DMA] * _SBUF,
        ],
    )(m, dst3, zer)
    return f2.reshape(2, Np, H)


def _node_step_body(nL, x, aggp, Wnx, Wna, bn0, W, b, Wzs, Wzd,
                    x_o, zs_o, zd_o):
    agg = aggp[0] + aggp[1]
    h = _mmb(x[...], Wnx[...]) + _mmb(agg, Wna[...]) + bn0[...]
    for i in range(nL):
        h = _mmb(h, W[i]) + b[i:i + 1, :]
    x_o[...] = h
    zs_o[...] = _mmb(h, Wzs[...])
    zd_o[...] = _mmb(h, Wzd[...])


def _tc_node_step(N, H, x, aggp, Wnx, Wna, bn0, W, b, Wzs, Wzd):
    grid = (N // _NBLK,)
    row = pl.BlockSpec((_NBLK, H), lambda i: (i, 0))
    part = pl.BlockSpec((2, _NBLK, H), lambda i: (0, i, 0))
    full = lambda a: pl.BlockSpec(a.shape, lambda i: (0,) * a.ndim)
    return pl.pallas_call(
        functools.partial(_node_step_body, W.shape[0]),
        grid=grid,
        in_specs=[row, part, full(Wnx), full(Wna), full(bn0), full(W),
                  full(b), full(Wzs), full(Wzd)],
        out_specs=[row, row, row],
        out_shape=[jax.ShapeDtypeStruct((N, H), F32)] * 3,
    )(x, aggp, Wnx, Wna, bn0, W, b, Wzs, Wzd)


def _node_fin_body(nL, dL, x, aggp, Wnx, Wna, bn0, W, b, Wd, bd, out_o):
    agg = aggp[0] + aggp[1]
    h = _mmb(x[...], Wnx[...]) + _mmb(agg, Wna[...]) + bn0[...]
    for i in range(nL):
        h = _mmb(h, W[i]) + b[i:i + 1, :]
    for i in range(dL):
        h = _mmb(h, Wd[i]) + bd[i:i + 1, :]
    out_o[...] = h


def _tc_node_fin(N, H, Dn, x, aggp, Wnx, Wna, bn0, W, b, Wd, bd):
    grid = (N // _NBLK,)
    row = pl.BlockSpec((_NBLK, H), lambda i: (i, 0))
    part = pl.BlockSpec((2, _NBLK, H), lambda i: (0, i, 0))
    full = lambda a: pl.BlockSpec(a.shape, lambda i: (0,) * a.ndim)
    return pl.pallas_call(
        functools.partial(_node_fin_body, W.shape[0], Wd.shape[0]),
        grid=grid,
        in_specs=[row, part, full(Wnx), full(Wna), full(bn0), full(W),
                  full(b), full(Wd), full(bd)],
        out_specs=pl.BlockSpec((_NBLK, Dn), lambda i: (i, 0)),
        out_shape=jax.ShapeDtypeStruct((N, Dn), F32),
    )(x, aggp, Wnx, Wna, bn0, W, b, Wd, bd)


def kernel(node_features_in, edge_features_in, edges_indexes,
           enc_n_W, enc_n_b, enc_e_W0, enc_e_b0, enc_e_W, enc_e_b,
           proc_e_W0, proc_e_b0, proc_e_W, proc_e_b,
           proc_n_W0, proc_n_b0, proc_n_W, proc_n_b, dec_W, dec_b):
    N, Dn = node_features_in.shape
    E, De = edge_features_in.shape
    H = enc_n_W.shape[-1]
    S = proc_e_W0.shape[0]
    src = edges_indexes[0]
    dst = edges_indexes[1]

    ncht = E // (_NW * _CHN)
    assert E == _NW * ncht * _CHN
    src3 = src.reshape(_NW, ncht, _CHN)
    dst3 = dst.reshape(_NW, ncht, _CHN)

    x, zs, zd = _tc_node_enc(N, H, node_features_in, enc_n_W, enc_n_b,
                             proc_e_W0[0, H:2 * H], proc_e_W0[0, 2 * H:3 * H])
    for s in range(S):
        gsum = _sc_gsum(zs, zd, src3, dst3)
        if s == 0:
            e = _tc_edge_enc0(E, H, edge_features_in, enc_e_W0,
                              enc_e_b0.reshape(1, H), enc_e_W, enc_e_b,
                              gsum, proc_e_W0[0, 0:H],
                              proc_e_b0[0].reshape(1, H), proc_e_W[0],
                              proc_e_b[0])
        else:
            e = _tc_edge_step(E, H, e, gsum, proc_e_W0[s, 0:H],
                              proc_e_b0[s].reshape(1, H), proc_e_W[s],
                              proc_e_b[s])
        aggp = _sc_scatter(N, e, dst3)
        if s + 1 < S:
            x, zs, zd = _tc_node_step(
                N, H, x, aggp, proc_n_W0[s, 0:H], proc_n_W0[s, H:2 * H],
                proc_n_b0[s].reshape(1, H), proc_n_W[s], proc_n_b[s],
                proc_e_W0[s + 1, H:2 * H], proc_e_W0[s + 1, 2 * H:3 * H])
        else:
            out = _tc_node_fin(
                N, H, Dn, x, aggp, proc_n_W0[s, 0:H], proc_n_W0[s, H:2 * H],
                proc_n_b0[s].reshape(1, H), proc_n_W[s], proc_n_b[s],
                dec_W, dec_b)
    return out


# speedup vs baseline: 1.2787x; 1.0102x over previous
import functools

import jax
import jax.numpy as jnp
from jax import lax
from jax.experimental import pallas as pl
from jax.experimental.pallas import tpu as pltpu
from jax.experimental.pallas import tpu_sc as plsc

F32 = jnp.float32
BF = jnp.bfloat16
_NC, _NS = 2, 16
_NW = _NC * _NS
_CH = 128
_NBLK = 2000
_EBLK = 16000


def _mmb(a, b):
    return jnp.dot(a.astype(BF), b.astype(BF), preferred_element_type=F32)


def _pad_n(N):
    q = _NS * 8
    return (N + q - 1) // q * q


def _node_enc_body(nL, nf, W, b, Wzs, Wzd, x_o, zs_o, zd_o):
    x = nf[...]
    for i in range(nL):
        x = _mmb(x, W[i]) + b[i:i + 1, :]
    x_o[...] = x
    zs_o[...] = _mmb(x, Wzs[...])
    zd_o[...] = _mmb(x, Wzd[...])


def _tc_node_enc(N, H, nf, W, b, Wzs, Wzd):
    grid = (N // _NBLK,)
    row = pl.BlockSpec((_NBLK, H), lambda i: (i, 0))
    full = lambda a: pl.BlockSpec(a.shape, lambda i: (0,) * a.ndim)
    return pl.pallas_call(
        functools.partial(_node_enc_body, W.shape[0]),
        grid=grid,
        in_specs=[pl.BlockSpec((_NBLK, nf.shape[1]), lambda i: (i, 0)),
                  full(W), full(b), full(Wzs), full(Wzd)],
        out_specs=[row, row, row],
        out_shape=[jax.ShapeDtypeStruct((N, H), F32)] * 3,
    )(nf, W, b, Wzs, Wzd)


def _edge_enc_body(nL, ef, W0, b0, W, b, e_o):
    e = _mmb(ef[...], W0[...]) + b0[...]
    for i in range(nL):
        e = _mmb(e, W[i]) + b[i:i + 1, :]
    e_o[...] = e


def _tc_edge_enc(E, H, ef, W0, b0, W, b):
    grid = (E // _EBLK,)
    full = lambda a: pl.BlockSpec(a.shape, lambda i: (0,) * a.ndim)
    return pl.pallas_call(
        functools.partial(_edge_enc_body, W.shape[0]),
        grid=grid,
        in_specs=[pl.BlockSpec((_EBLK, ef.shape[1]), lambda i: (i, 0)),
                  full(W0), full(b0), full(W), full(b)],
        out_specs=pl.BlockSpec((_EBLK, H), lambda i: (i, 0)),
        out_shape=jax.ShapeDtypeStruct((E, H), F32),
    )(ef, W0, b0, W, b)


def _edge_enc0_body(nLe, nL, ef, W0, b0, W, b, gsum, W0e, b0s, Ws, bs, m_o):
    e = _mmb(ef[...], W0[...]) + b0[...]
    for i in range(nLe):
        e = _mmb(e, W[i]) + b[i:i + 1, :]
    m = _mmb(e, W0e[...]) + gsum[...] + b0s[...]
    for i in range(nL):
        m = _mmb(m, Ws[i]) + bs[i:i + 1, :]
    m_o[...] = m


def _tc_edge_enc0(E, H, ef, W0, b0, W, b, gsum, W0e, b0s, Ws, bs):
    grid = (E // _EBLK,)
    row = pl.BlockSpec((_EBLK, H), lambda i: (i, 0))
    full = lambda a: pl.BlockSpec(a.shape, lambda i: (0,) * a.ndim)
    return pl.pallas_call(
        functools.partial(_edge_enc0_body, W.shape[0], Ws.shape[0]),
        grid=grid,
        in_specs=[pl.BlockSpec((_EBLK, ef.shape[1]), lambda i: (i, 0)),
                  full(W0), full(b0), full(W), full(b), row,
                  full(W0e), full(b0s), full(Ws), full(bs)],
        out_specs=row,
        out_shape=jax.ShapeDtypeStruct((E, H), F32),
    )(ef, W0, b0, W, b, gsum, W0e, b0s, Ws, bs)


_CHN = 80
_NBUF = 4


def _gsum_body(H, ncht, zs_h, zd_h, src3_h, dst3_h, g_out, junk_out,
               idxs, idxd, rows_a, rows_b, sem_a, sem_b, sem_w):
    cid = lax.axis_index("c")
    sid = lax.axis_index("s")
    wid = sid * _NC + cid
    ebase = wid * (ncht * _CHN)
    pltpu.sync_copy(src3_h.at[wid], idxs)
    pltpu.sync_copy(dst3_h.at[wid], idxd)

    def vadd(jp):
        @pl.loop(0, _CHN)
        def _(r):
            for c8 in range(H // 16):
                rows_a[jp, r, pl.ds(c8 * 16, 16)] = (
                    rows_a[jp, r, pl.ds(c8 * 16, 16)]
                    + rows_b[jp, r, pl.ds(c8 * 16, 16)])

    def gather(c, j):
        pltpu.async_copy(zs_h.at[idxs.at[c]], rows_a.at[j], sem_a[j])
        pltpu.async_copy(zd_h.at[idxd.at[c]], rows_b.at[j], sem_b[j])

    def wait_g(c, j):
        pltpu.make_async_copy(zs_h.at[idxs.at[c]], rows_a.at[j], sem_a[j]).wait()
        pltpu.make_async_copy(zd_h.at[idxd.at[c]], rows_b.at[j], sem_b[j]).wait()

    def wb(c, j):
        pltpu.async_copy(rows_a.at[j], g_out.at[pl.ds(ebase + c * _CHN, _CHN)],
                         sem_w[j])

    def wait_w(j):
        pltpu.make_async_copy(zs_h.at[pl.ds(0, _CHN)], rows_a.at[j],
                              sem_w[j]).wait()

    for j in range(_NBUF):
        pltpu.async_copy(rows_a.at[j], junk_out, sem_w[j])

    nloop = (ncht - 2) // _NBUF

    @pl.loop(0, nloop)
    def _(g):
        for j in range(_NBUF):
            c = g * _NBUF + j
            jp = (j + _NBUF - 2) % _NBUF
            wait_w(j)
            gather(c, j)

            @pl.when(c >= 2)
            def _():
                wait_g(c - 2, jp)
                vadd(jp)
                wb(c - 2, jp)

    for k in range(nloop * _NBUF, ncht):
        j = k % _NBUF
        wait_w(j)
        gather(k, j)
        jp = (j + _NBUF - 2) % _NBUF
        wait_g(k - 2, jp)
        vadd(jp)
        wb(k - 2, jp)
    for k in range(ncht - 2, ncht):
        jlast = k % _NBUF
        wait_g(k, jlast)
        vadd(jlast)
        wb(k, jlast)
    for j in range(_NBUF):
        wait_w(j)


def _sc_gsum(zs, zd, src3, dst3):
    NW, ncht, CHN = src3.shape
    E = NW * ncht * CHN
    H = zs.shape[1]
    mesh = plsc.VectorSubcoreMesh(core_axis_name="c", subcore_axis_name="s")
    g, _ = pl.kernel(
        functools.partial(_gsum_body, H, ncht),
        out_type=(jax.ShapeDtypeStruct((E, H), F32),
                  jax.ShapeDtypeStruct((_CHN, H), F32)),
        mesh=mesh,
        scratch_types=[
            pltpu.VMEM((ncht, CHN), jnp.int32),
            pltpu.VMEM((ncht, CHN), jnp.int32),
            pltpu.VMEM((_NBUF, _CHN, H), F32),
            pltpu.VMEM((_NBUF, _CHN, H), F32),
            [pltpu.SemaphoreType.DMA] * _NBUF,
            [pltpu.SemaphoreType.DMA] * _NBUF,
            [pltpu.SemaphoreType.DMA] * _NBUF,
        ],
    )(zs, zd, src3, dst3)
    return g


def _edge_step_body(nL, e, gsum, W0e, b0, W, b, m_o):
    m = _mmb(e[...], W0e[...]) + gsum[...] + b0[...]
    for i in range(nL):
        m = _mmb(m, W[i]) + b[i:i + 1, :]
    m_o[...] = m


def _tc_edge_step(E, H, e, gsum, W0e, b0, W, b):
    grid = (E // _EBLK,)
    row = pl.BlockSpec((_EBLK, H), lambda i: (i, 0))
    full = lambda a: pl.BlockSpec(a.shape, lambda i: (0,) * a.ndim)
    return pl.pallas_call(
        functools.partial(_edge_step_body, W.shape[0]),
        grid=grid,
        in_specs=[row, row, full(W0e), full(b0), full(W), full(b)],
        out_specs=row,
        out_shape=jax.ShapeDtypeStruct((E, H), F32),
    )(e, gsum, W0e, b0, W, b)


_SBUF = 3


def _scat_body(Np, H, ncht, m_h, dst3_h, zer_h, f_out,
               idxd, rows, f_sh, sem_l):
    cid = lax.axis_index("c")
    sid = lax.axis_index("s")
    wid = sid * _NC + cid
    rpt = Np // _NS
    ebase = wid * (ncht * _CHN)
    pltpu.sync_copy(dst3_h.at[wid], idxd)
    pltpu.sync_copy(zer_h, f_sh.at[pl.ds(sid * rpt, rpt)])
    plsc.subcore_barrier()

    def load(c, j):
        pltpu.async_copy(m_h.at[pl.ds(ebase + c * _CHN, _CHN)], rows.at[j],
                         sem_l[j])

    def wait_l(j):
        pltpu.make_async_copy(m_h.at[pl.ds(0, _CHN)], rows.at[j],
                              sem_l[j]).wait()

    load(0, 0)
    nloop = ncht // _SBUF

    @pl.loop(0, nloop)
    def _(g):
        for j in range(_SBUF):
            c = g * _SBUF + j
            jn = (j + 1) % _SBUF
            wait_l(j)
            cn = jnp.minimum(c + 1, ncht - 1)
            load(cn, jn)
            pltpu.sync_copy(rows.at[j], f_sh.at[idxd.at[c]], add=True)

    for k in range(nloop * _SBUF, ncht):
        j = k % _SBUF
        wait_l(j)
        load(min(k + 1, ncht - 1), (j + 1) % _SBUF)
        pltpu.sync_copy(rows.at[j], f_sh.at[idxd.at[k]], add=True)
    wait_l(ncht % _SBUF)

    plsc.subcore_barrier()
    row0 = cid * Np + sid * rpt
    pltpu.sync_copy(f_sh.at[pl.ds(sid * rpt, rpt)], f_out.at[pl.ds(row0, rpt)])


def _sc_scatter(N, m, dst3):
    NW, ncht, CHN = dst3.shape
    E, H = m.shape
    Np = _pad_n(N)
    rpt = Np // _NS
    zer = jnp.zeros((rpt, H), F32)
    mesh = plsc.VectorSubcoreMesh(core_axis_name="c", subcore_axis_name="s")
    f2 = pl.kernel(
        functools.partial(_scat_body, Np, H, ncht),
        out_type=jax.ShapeDtypeStruct((2 * Np, H), F32),
        mesh=mesh,
        scratch_types=[
            pltpu.VMEM((ncht, CHN), jnp.int32),
            pltpu.VMEM((_SBUF, _CHN, H), F32),
            pltpu.VMEM_SHARED((Np, H), F32),
            [pltpu.SemaphoreType.DMA] * _SBUF,
        ],
    )(m, dst3, zer)
    return f2.reshape(2, Np, H)


def _node_step_body(nL, x, aggp, Wnx, Wna, bn0, W, b, Wzs, Wzd,
                    x_o, zs_o, zd_o):
    agg = aggp[0] + aggp[1]
    h = _mmb(x[...], Wnx[...]) + _mmb(agg, Wna[...]) + bn0[...]
    for i in range(nL):
        h = _mmb(h, W[i]) + b[i:i + 1, :]
    x_o[...] = h
    zs_o[...] = _mmb(h, Wzs[...])
    zd_o[...] = _mmb(h, Wzd[...])


def _tc_node_step(N, H, x, aggp, Wnx, Wna, bn0, W, b, Wzs, Wzd):
    grid = (N // _NBLK,)
    row = pl.BlockSpec((_NBLK, H), lambda i: (i, 0))
    part = pl.BlockSpec((2, _NBLK, H), lambda i: (0, i, 0))
    full = lambda a: pl.BlockSpec(a.shape, lambda i: (0,) * a.ndim)
    return pl.pallas_call(
        functools.partial(_node_step_body, W.shape[0]),
        grid=grid,
        in_specs=[row, part, full(Wnx), full(Wna), full(bn0), full(W),
                  full(b), full(Wzs), full(Wzd)],
        out_specs=[row, row, row],
        out_shape=[jax.ShapeDtypeStruct((N, H), F32)] * 3,
    )(x, aggp, Wnx, Wna, bn0, W, b, Wzs, Wzd)


def _node_fin_body(nL, dL, x, aggp, Wnx, Wna, bn0, W, b, Wd, bd, out_o):
    agg = aggp[0] + aggp[1]
    h = _mmb(x[...], Wnx[...]) + _mmb(agg, Wna[...]) + bn0[...]
    for i in range(nL):
        h = _mmb(h, W[i]) + b[i:i + 1, :]
    for i in range(dL):
        h = _mmb(h, Wd[i]) + bd[i:i + 1, :]
    out_o[...] = h


def _tc_node_fin(N, H, Dn, x, aggp, Wnx, Wna, bn0, W, b, Wd, bd):
    grid = (N // _NBLK,)
    row = pl.BlockSpec((_NBLK, H), lambda i: (i, 0))
    part = pl.BlockSpec((2, _NBLK, H), lambda i: (0, i, 0))
    full = lambda a: pl.BlockSpec(a.shape, lambda i: (0,) * a.ndim)
    return pl.pallas_call(
        functools.partial(_node_fin_body, W.shape[0], Wd.shape[0]),
        grid=grid,
        in_specs=[row, part, full(Wnx), full(Wna), full(bn0), full(W),
                  full(b), full(Wd), full(bd)],
        out_specs=pl.BlockSpec((_NBLK, Dn), lambda i: (i, 0)),
        out_shape=jax.ShapeDtypeStruct((N, Dn), F32),
    )(x, aggp, Wnx, Wna, bn0, W, b, Wd, bd)


def kernel(node_features_in, edge_features_in, edges_indexes,
           enc_n_W, enc_n_b, enc_e_W0, enc_e_b0, enc_e_W, enc_e_b,
           proc_e_W0, proc_e_b0, proc_e_W, proc_e_b,
           proc_n_W0, proc_n_b0, proc_n_W, proc_n_b, dec_W, dec_b):
    N, Dn = node_features_in.shape
    E, De = edge_features_in.shape
    H = enc_n_W.shape[-1]
    S = proc_e_W0.shape[0]
    src = edges_indexes[0]
    dst = edges_indexes[1]

    ncht = E // (_NW * _CHN)
    assert E == _NW * ncht * _CHN
    src3 = src.reshape(_NW, ncht, _CHN)
    dst3 = dst.reshape(_NW, ncht, _CHN)

    x, zs, zd = _tc_node_enc(N, H, node_features_in, enc_n_W, enc_n_b,
                             proc_e_W0[0, H:2 * H], proc_e_W0[0, 2 * H:3 * H])
    for s in range(S):
        gsum = _sc_gsum(zs, zd, src3, dst3)
        if s == 0:
            e = _tc_edge_enc0(E, H, edge_features_in, enc_e_W0,
                              enc_e_b0.reshape(1, H), enc_e_W, enc_e_b,
                              gsum, proc_e_W0[0, 0:H],
                              proc_e_b0[0].reshape(1, H), proc_e_W[0],
                              proc_e_b[0])
        else:
            e = _tc_edge_step(E, H, e, gsum, proc_e_W0[s, 0:H],
                              proc_e_b0[s].reshape(1, H), proc_e_W[s],
                              proc_e_b[s])
        aggp = _sc_scatter(N, e, dst3)
        if s + 1 < S:
            x, zs, zd = _tc_node_step(
                N, H, x, aggp, proc_n_W0[s, 0:H], proc_n_W0[s, H:2 * H],
                proc_n_b0[s].reshape(1, H), proc_n_W[s], proc_n_b[s],
                proc_e_W0[s + 1, H:2 * H], proc_e_W0[s + 1, 2 * H:3 * H])
        else:
            out = _tc_node_fin(
                N, H, Dn, x, aggp, proc_n_W0[s, 0:H], proc_n_W0[s, H:2 * H],
                proc_n_b0[s].reshape(1, H), proc_n_W[s], proc_n_b[s],
                dec_W, dec_b)
    return out


# speedup vs baseline: 1.2792x; 1.0004x over previous
import functools

import jax
import jax.numpy as jnp
from jax import lax
from jax.experimental import pallas as pl
from jax.experimental.pallas import tpu as pltpu
from jax.experimental.pallas import tpu_sc as plsc

F32 = jnp.float32
BF = jnp.bfloat16
_NC, _NS = 2, 16
_NW = _NC * _NS
_CH = 128
_NBLK = 2000
_EBLK = 16000


def _mmb(a, b):
    return jnp.dot(a.astype(BF), b.astype(BF), preferred_element_type=F32)


def _pad_n(N):
    q = _NS * 8
    return (N + q - 1) // q * q


def _node_enc_body(nL, nf, W, b, Wzs, Wzd, x_o, zs_o, zd_o):
    x = nf[...]
    for i in range(nL):
        x = _mmb(x, W[i]) + b[i:i + 1, :]
    x_o[...] = x
    zs_o[...] = _mmb(x, Wzs[...])
    zd_o[...] = _mmb(x, Wzd[...])


def _tc_node_enc(N, H, nf, W, b, Wzs, Wzd):
    grid = (N // _NBLK,)
    row = pl.BlockSpec((_NBLK, H), lambda i: (i, 0))
    full = lambda a: pl.BlockSpec(a.shape, lambda i: (0,) * a.ndim)
    return pl.pallas_call(
        functools.partial(_node_enc_body, W.shape[0]),
        grid=grid,
        in_specs=[pl.BlockSpec((_NBLK, nf.shape[1]), lambda i: (i, 0)),
                  full(W), full(b), full(Wzs), full(Wzd)],
        out_specs=[row, row, row],
        out_shape=[jax.ShapeDtypeStruct((N, H), F32)] * 3,
    )(nf, W, b, Wzs, Wzd)


def _edge_enc_body(nL, ef, W0, b0, W, b, e_o):
    e = _mmb(ef[...], W0[...]) + b0[...]
    for i in range(nL):
        e = _mmb(e, W[i]) + b[i:i + 1, :]
    e_o[...] = e


def _tc_edge_enc(E, H, ef, W0, b0, W, b):
    grid = (E // _EBLK,)
    full = lambda a: pl.BlockSpec(a.shape, lambda i: (0,) * a.ndim)
    return pl.pallas_call(
        functools.partial(_edge_enc_body, W.shape[0]),
        grid=grid,
        in_specs=[pl.BlockSpec((_EBLK, ef.shape[1]), lambda i: (i, 0)),
                  full(W0), full(b0), full(W), full(b)],
        out_specs=pl.BlockSpec((_EBLK, H), lambda i: (i, 0)),
        out_shape=jax.ShapeDtypeStruct((E, H), F32),
    )(ef, W0, b0, W, b)


def _edge_enc0_body(nLe, nL, ef, W0, b0, W, b, gsum, W0e, b0s, Ws, bs, m_o):
    e = _mmb(ef[...], W0[...]) + b0[...]
    for i in range(nLe):
        e = _mmb(e, W[i]) + b[i:i + 1, :]
    m = _mmb(e, W0e[...]) + gsum[...] + b0s[...]
    for i in range(nL):
        m = _mmb(m, Ws[i]) + bs[i:i + 1, :]
    m_o[...] = m


def _tc_edge_enc0(E, H, ef, W0, b0, W, b, gsum, W0e, b0s, Ws, bs):
    grid = (E // _EBLK,)
    row = pl.BlockSpec((_EBLK, H), lambda i: (i, 0))
    full = lambda a: pl.BlockSpec(a.shape, lambda i: (0,) * a.ndim)
    return pl.pallas_call(
        functools.partial(_edge_enc0_body, W.shape[0], Ws.shape[0]),
        grid=grid,
        in_specs=[pl.BlockSpec((_EBLK, ef.shape[1]), lambda i: (i, 0)),
                  full(W0), full(b0), full(W), full(b), row,
                  full(W0e), full(b0s), full(Ws), full(bs)],
        out_specs=row,
        out_shape=jax.ShapeDtypeStruct((E, H), F32),
    )(ef, W0, b0, W, b, gsum, W0e, b0s, Ws, bs)


_CHN = 80
_NBUF = 4


def _gsum_body(H, ncht, zs_h, zd_h, src3_h, dst3_h, g_out, junk_out,
               idxs, idxd, rows_a, rows_b, sem_a, sem_b, sem_w):
    cid = lax.axis_index("c")
    sid = lax.axis_index("s")
    wid = sid * _NC + cid
    ebase = wid * (ncht * _CHN)
    pltpu.sync_copy(src3_h.at[wid], idxs)
    pltpu.sync_copy(dst3_h.at[wid], idxd)

    def vadd(jp):
        @pl.loop(0, _CHN)
        def _(r):
            for c8 in range(H // 16):
                rows_a[jp, r, pl.ds(c8 * 16, 16)] = (
                    rows_a[jp, r, pl.ds(c8 * 16, 16)]
                    + rows_b[jp, r, pl.ds(c8 * 16, 16)])

    def gather(c, j):
        pltpu.async_copy(zs_h.at[idxs.at[c]], rows_a.at[j], sem_a[j])
        pltpu.async_copy(zd_h.at[idxd.at[c]], rows_b.at[j], sem_b[j])

    def wait_g(c, j):
        pltpu.make_async_copy(zs_h.at[idxs.at[c]], rows_a.at[j], sem_a[j]).wait()
        pltpu.make_async_copy(zd_h.at[idxd.at[c]], rows_b.at[j], sem_b[j]).wait()

    def wb(c, j):
        pltpu.async_copy(rows_a.at[j], g_out.at[pl.ds(ebase + c * _CHN, _CHN)],
                         sem_w[j])

    def wait_w(j):
        pltpu.make_async_copy(zs_h.at[pl.ds(0, _CHN)], rows_a.at[j],
                              sem_w[j]).wait()

    for j in range(_NBUF):
        pltpu.async_copy(rows_a.at[j], junk_out, sem_w[j])

    nloop = (ncht - 2) // _NBUF

    @pl.loop(0, nloop)
    def _(g):
        for j in range(_NBUF):
            c = g * _NBUF + j
            jp = (j + _NBUF - 2) % _NBUF
            wait_w(j)
            gather(c, j)

            @pl.when(c >= 2)
            def _():
                wait_g(c - 2, jp)
                vadd(jp)
                wb(c - 2, jp)

    for k in range(nloop * _NBUF, ncht):
        j = k % _NBUF
        wait_w(j)
        gather(k, j)
        jp = (j + _NBUF - 2) % _NBUF
        wait_g(k - 2, jp)
        vadd(jp)
        wb(k - 2, jp)
    for k in range(ncht - 2, ncht):
        jlast = k % _NBUF
        wait_g(k, jlast)
        vadd(jlast)
        wb(k, jlast)
    for j in range(_NBUF):
        wait_w(j)


def _sc_gsum(zs, zd, src3, dst3):
    NW, ncht, CHN = src3.shape
    E = NW * ncht * CHN
    H = zs.shape[1]
    mesh = plsc.VectorSubcoreMesh(core_axis_name="c", subcore_axis_name="s")
    g, _ = pl.kernel(
        functools.partial(_gsum_body, H, ncht),
        out_type=(jax.ShapeDtypeStruct((E, H), F32),
                  jax.ShapeDtypeStruct((_CHN, H), F32)),
        mesh=mesh,
        scratch_types=[
            pltpu.VMEM((ncht, CHN), jnp.int32),
            pltpu.VMEM((ncht, CHN), jnp.int32),
            pltpu.VMEM((_NBUF, _CHN, H), F32),
            pltpu.VMEM((_NBUF, _CHN, H), F32),
            [pltpu.SemaphoreType.DMA] * _NBUF,
            [pltpu.SemaphoreType.DMA] * _NBUF,
            [pltpu.SemaphoreType.DMA] * _NBUF,
        ],
    )(zs, zd, src3, dst3)
    return g


def _edge_step_body(nL, e, gsum, W0e, b0, W, b, m_o):
    m = _mmb(e[...], W0e[...]) + gsum[...] + b0[...]
    for i in range(nL):
        m = _mmb(m, W[i]) + b[i:i + 1, :]
    m_o[...] = m


def _tc_edge_step(E, H, e, gsum, W0e, b0, W, b):
    grid = (E // _EBLK,)
    row = pl.BlockSpec((_EBLK, H), lambda i: (i, 0))
    full = lambda a: pl.BlockSpec(a.shape, lambda i: (0,) * a.ndim)
    return pl.pallas_call(
        functools.partial(_edge_step_body, W.shape[0]),
        grid=grid,
        in_specs=[row, row, full(W0e), full(b0), full(W), full(b)],
        out_specs=row,
        out_shape=jax.ShapeDtypeStruct((E, H), F32),
    )(e, gsum, W0e, b0, W, b)


_SBUF = 3


def _scat_body(Np, H, ncht, m_h, dst3_h, zer_h, f_out,
               idxd, rows, f_sh, sem_l, sem_s):
    cid = lax.axis_index("c")
    sid = lax.axis_index("s")
    wid = sid * _NC + cid
    rpt = Np // _NS
    ebase = wid * (ncht * _CHN)
    pltpu.sync_copy(dst3_h.at[wid], idxd)
    pltpu.sync_copy(zer_h, f_sh.at[pl.ds(sid * rpt, rpt)])
    plsc.subcore_barrier()

    def load(c, j):
        pltpu.async_copy(m_h.at[pl.ds(ebase + c * _CHN, _CHN)], rows.at[j],
                         sem_l[j])

    def wait_l(j):
        pltpu.make_async_copy(m_h.at[pl.ds(0, _CHN)], rows.at[j],
                              sem_l[j]).wait()

    def scat(c, j):
        pltpu.async_copy(rows.at[j], f_sh.at[idxd.at[c]], sem_s[j], add=True)

    def wait_s(j):
        pltpu.make_async_copy(rows.at[j], f_sh.at[idxd.at[0]],
                              sem_s[j]).wait()

    load(0, 0)
    nloop = ncht // _SBUF

    @pl.loop(0, nloop)
    def _(g):
        for j in range(_SBUF):
            c = g * _SBUF + j
            jn = (j + 1) % _SBUF
            wait_l(j)
            scat(c, j)

            @pl.when(c >= 2)
            def _():
                wait_s(jn)
            cn = jnp.minimum(c + 1, ncht - 1)
            load(cn, jn)

    for k in range(nloop * _SBUF, ncht):
        j = k % _SBUF
        jn = (j + 1) % _SBUF
        wait_l(j)
        scat(k, j)
        wait_s(jn)
        load(min(k + 1, ncht - 1), jn)
    wait_l(ncht % _SBUF)
    wait_s((ncht - 2) % _SBUF)
    wait_s((ncht - 1) % _SBUF)

    plsc.subcore_barrier()
    row0 = cid * Np + sid * rpt
    pltpu.sync_copy(f_sh.at[pl.ds(sid * rpt, rpt)], f_out.at[pl.ds(row0, rpt)])


def _sc_scatter(N, m, dst3):
    NW, ncht, CHN = dst3.shape
    E, H = m.shape
    Np = _pad_n(N)
    rpt = Np // _NS
    zer = jnp.zeros((rpt, H), F32)
    mesh = plsc.VectorSubcoreMesh(core_axis_name="c", subcore_axis_name="s")
    f2 = pl.kernel(
        functools.partial(_scat_body, Np, H, ncht),
        out_type=jax.ShapeDtypeStruct((2 * Np, H), F32),
        mesh=mesh,
        scratch_types=[
            pltpu.VMEM((ncht, CHN), jnp.int32),
            pltpu.VMEM((_SBUF, _CHN, H), F32),
            pltpu.VMEM_SHARED((Np, H), F32),
            [pltpu.SemaphoreType.DMA] * _SBUF,
            [pltpu.SemaphoreType.DMA] * _SBUF,
        ],
    )(m, dst3, zer)
    return f2.reshape(2, Np, H)


def _node_step_body(nL, x, aggp, Wnx, Wna, bn0, W, b, Wzs, Wzd,
                    x_o, zs_o, zd_o):
    agg = aggp[0] + aggp[1]
    h = _mmb(x[...], Wnx[...]) + _mmb(agg, Wna[...]) + bn0[...]
    for i in range(nL):
        h = _mmb(h, W[i]) + b[i:i + 1, :]
    x_o[...] = h
    zs_o[...] = _mmb(h, Wzs[...])
    zd_o[...] = _mmb(h, Wzd[...])


def _tc_node_step(N, H, x, aggp, Wnx, Wna, bn0, W, b, Wzs, Wzd):
    grid = (N // _NBLK,)
    row = pl.BlockSpec((_NBLK, H), lambda i: (i, 0))
    part = pl.BlockSpec((2, _NBLK, H), lambda i: (0, i, 0))
    full = lambda a: pl.BlockSpec(a.shape, lambda i: (0,) * a.ndim)
    return pl.pallas_call(
        functools.partial(_node_step_body, W.shape[0]),
        grid=grid,
        in_specs=[row, part, full(Wnx), full(Wna), full(bn0), full(W),
                  full(b), full(Wzs), full(Wzd)],
        out_specs=[row, row, row],
        out_shape=[jax.ShapeDtypeStruct((N, H), F32)] * 3,
    )(x, aggp, Wnx, Wna, bn0, W, b, Wzs, Wzd)


def _node_fin_body(nL, dL, x, aggp, Wnx, Wna, bn0, W, b, Wd, bd, out_o):
    agg = aggp[0] + aggp[1]
    h = _mmb(x[...], Wnx[...]) + _mmb(agg, Wna[...]) + bn0[...]
    for i in range(nL):
        h = _mmb(h, W[i]) + b[i:i + 1, :]
    for i in range(dL):
        h = _mmb(h, Wd[i]) + bd[i:i + 1, :]
    out_o[...] = h


def _tc_node_fin(N, H, Dn, x, aggp, Wnx, Wna, bn0, W, b, Wd, bd):
    grid = (N // _NBLK,)
    row = pl.BlockSpec((_NBLK, H), lambda i: (i, 0))
    part = pl.BlockSpec((2, _NBLK, H), lambda i: (0, i, 0))
    full = lambda a: pl.BlockSpec(a.shape, lambda i: (0,) * a.ndim)
    return pl.pallas_call(
        functools.partial(_node_fin_body, W.shape[0], Wd.shape[0]),
        grid=grid,
        in_specs=[row, part, full(Wnx), full(Wna), full(bn0), full(W),
                  full(b), full(Wd), full(bd)],
        out_specs=pl.BlockSpec((_NBLK, Dn), lambda i: (i, 0)),
        out_shape=jax.ShapeDtypeStruct((N, Dn), F32),
    )(x, aggp, Wnx, Wna, bn0, W, b, Wd, bd)


def kernel(node_features_in, edge_features_in, edges_indexes,
           enc_n_W, enc_n_b, enc_e_W0, enc_e_b0, enc_e_W, enc_e_b,
           proc_e_W0, proc_e_b0, proc_e_W, proc_e_b,
           proc_n_W0, proc_n_b0, proc_n_W, proc_n_b, dec_W, dec_b):
    N, Dn = node_features_in.shape
    E, De = edge_features_in.shape
    H = enc_n_W.shape[-1]
    S = proc_e_W0.shape[0]
    src = edges_indexes[0]
    dst = edges_indexes[1]

    ncht = E // (_NW * _CHN)
    assert E == _NW * ncht * _CHN
    src3 = src.reshape(_NW, ncht, _CHN)
    dst3 = dst.reshape(_NW, ncht, _CHN)

    x, zs, zd = _tc_node_enc(N, H, node_features_in, enc_n_W, enc_n_b,
                             proc_e_W0[0, H:2 * H], proc_e_W0[0, 2 * H:3 * H])
    for s in range(S):
        gsum = _sc_gsum(zs, zd, src3, dst3)
        if s == 0:
            e = _tc_edge_enc0(E, H, edge_features_in, enc_e_W0,
                              enc_e_b0.reshape(1, H), enc_e_W, enc_e_b,
                              gsum, proc_e_W0[0, 0:H],
                              proc_e_b0[0].reshape(1, H), proc_e_W[0],
                              proc_e_b[0])
        else:
            e = _tc_edge_step(E, H, e, gsum, proc_e_W0[s, 0:H],
                              proc_e_b0[s].reshape(1, H), proc_e_W[s],
                              proc_e_b[s])
        aggp = _sc_scatter(N, e, dst3)
        if s + 1 < S:
            x, zs, zd = _tc_node_step(
                N, H, x, aggp, proc_n_W0[s, 0:H], proc_n_W0[s, H:2 * H],
                proc_n_b0[s].reshape(1, H), proc_n_W[s], proc_n_b[s],
                proc_e_W0[s + 1, H:2 * H], proc_e_W0[s + 1, 2 * H:3 * H])
        else:
            out = _tc_node_fin(
                N, H, Dn, x, aggp, proc_n_W0[s, 0:H], proc_n_W0[s, H:2 * H],
                proc_n_b0[s].reshape(1, H), proc_n_W[s], proc_n_b[s],
                dec_W, dec_b)
    return out
